# baseline (speedup 1.0000x reference)
import functools

import jax
import jax.numpy as jnp
from jax import lax
from jax.experimental import pallas as pl
from jax.experimental.pallas import tpu as pltpu
from jax.experimental.pallas import tpu_sc as plsc

EMB = 64
SCALE = 8.0
NC = 2
NS = 16
L = 16
NW = NC * NS
CHUNK = 128
ET = EMB // 8


@functools.lru_cache(maxsize=None)
def _make(n_s):
    nb = 4 if n_s % 4 == 0 else (2 if n_s % 2 == 0 else 1)
    mesh = plsc.VectorSubcoreMesh(
        core_axis_name="c", subcore_axis_name="s",
        num_cores=NC, num_subcores=NS)

    def body(tok_hbm, table_hbm, out_hbm, tok_v, idx_v, *bufs):
        rows = bufs[:nb]
        fmaj = bufs[nb:2 * nb]
        gi = bufs[2 * nb:3 * nb]
        wo = bufs[3 * nb:]
        wid = lax.axis_index("s") * NC + lax.axis_index("c")
        pltpu.sync_copy(tok_hbm.at[pl.ds(wid * CHUNK, CHUNK)], tok_v)

        lane = lax.broadcasted_iota(jnp.int32, (L,), 0)
        row_idx = [lane + (j * L) for j in range(CHUNK // L)]

        @pl.loop(0, n_s)
        def _(s):
            col = jnp.full((L,), s, jnp.int32)
            for j in range(CHUNK // L):
                idx_v[s, pl.ds(j * L, L)] = plsc.load_gather(
                    tok_v, [row_idx[j], col])

        def fire_gather(s, b):
            pltpu.async_copy(table_hbm.at[idx_v.at[s]], rows[b], gi[b])

        def drain_gather(s, b):
            pltpu.make_async_copy(
                table_hbm.at[idx_v.at[s]], rows[b], gi[b]).wait()

        def fire_out(s, b):
            pltpu.async_copy(fmaj[b], out_hbm.at[s, :, wid], wo[b])

        def drain_out(s, b):
            pltpu.make_async_copy(
                fmaj[b], out_hbm.at[s, :, wid], wo[b]).wait()

        def transpose(b):
            r, f = rows[b], fmaj[b]

            @pl.loop(0, EMB)
            def _(e):
                col = jnp.full((L,), e, jnp.int32)
                for j in range(CHUNK // L):
                    v = plsc.load_gather(r, [row_idx[j], col])
                    f[e // 8, e % 8, pl.ds(j * L, L)] = v * SCALE

        for b in range(nb):
            fire_gather(b, b)

        @pl.loop(0, n_s - nb, step=nb)
        def _(s0):
            for b in range(nb):
                drain_gather(s0 + b, b)
                transpose(b)
                fire_out(s0 + b, b)
                fire_gather(s0 + nb + b, b)
            for b in range(nb):
                drain_out(s0 + b, b)

        for b in range(nb):
            drain_gather(n_s - nb + b, b)
            transpose(b)
            fire_out(n_s - nb + b, b)
        for b in range(nb):
            drain_out(n_s - nb + b, b)

    return pl.kernel(
        body,
        out_type=jax.ShapeDtypeStruct((n_s, ET, NW, 8, CHUNK), jnp.float32),
        mesh=mesh,
        compiler_params=pltpu.CompilerParams(
            use_tc_tiling_on_sc=False, needs_layout_passes=False),
        scratch_types=(
            [pltpu.VMEM((CHUNK, n_s), jnp.int32),
             pltpu.VMEM((n_s, CHUNK), jnp.int32)]
            + [pltpu.VMEM((CHUNK, EMB), jnp.float32) for _ in range(nb)]
            + [pltpu.VMEM((ET, 8, CHUNK), jnp.float32) for _ in range(nb)]
            + [pltpu.SemaphoreType.DMA for _ in range(2 * nb)]
        ),
    )


def kernel(tokens, table):
    bt, n_s = tokens.shape
    out5 = _make(n_s)(tokens.astype(jnp.int32), table)
    return out5.transpose(2, 4, 0, 1, 3).reshape(bt, n_s, EMB)


# speedup vs baseline: 1.7196x; 1.7196x over previous
import functools

import jax
import jax.numpy as jnp
from jax import lax
from jax.experimental import pallas as pl
from jax.experimental.pallas import tpu as pltpu
from jax.experimental.pallas import tpu_sc as plsc

EMB = 64
SCALE = 8.0
NC = 2
NS = 16
L = 16
NW = NC * NS
CHUNK = 128
ET = EMB // 8


@functools.lru_cache(maxsize=None)
def _make(n_s):
    nb = 4 if n_s % 4 == 0 else (2 if n_s % 2 == 0 else 1)
    mesh = plsc.VectorSubcoreMesh(
        core_axis_name="c", subcore_axis_name="s",
        num_cores=NC, num_subcores=NS)

    def body(tok_hbm, table_hbm, out_hbm, tok_v, idx_v, *bufs):
        rows = bufs[:nb]
        fmaj = bufs[nb:2 * nb]
        gi = bufs[2 * nb:3 * nb]
        wo = bufs[3 * nb:]
        wid = lax.axis_index("s") * NC + lax.axis_index("c")
        pltpu.sync_copy(tok_hbm.at[pl.ds(wid * CHUNK, CHUNK)], tok_v)

        lane = lax.broadcasted_iota(jnp.int32, (L,), 0)
        row_idx = [lane + (j * L) for j in range(CHUNK // L)]

        @pl.loop(0, n_s)
        def _(s):
            col = jnp.full((L,), s, jnp.int32)
            for j in range(CHUNK // L):
                idx_v[s, pl.ds(j * L, L)] = plsc.load_gather(
                    tok_v, [row_idx[j], col])

        def fire_gather(s, b):
            pltpu.async_copy(table_hbm.at[idx_v.at[s]], rows[b], gi[b])

        def drain_gather(s, b):
            pltpu.make_async_copy(
                table_hbm.at[idx_v.at[s]], rows[b], gi[b]).wait()

        def src_out(b):
            return fmaj[b].at[:, :, pl.ds(0, CHUNK)]

        def fire_out(s, b):
            pltpu.async_copy(src_out(b), out_hbm.at[s, :, wid], wo[b])

        def drain_out(s, b):
            pltpu.make_async_copy(
                src_out(b), out_hbm.at[s, :, wid], wo[b]).wait()

        ehi = [(2 * j) + (lane >= 8).astype(jnp.int32)
               for j in range(EMB // L)]
        elo = [lane & 7 for _ in range(EMB // L)]

        def transpose(b):
            r, f = rows[b], fmaj[b]

            @pl.loop(0, CHUNK)
            def _(t):
                tv = jnp.full((L,), t, jnp.int32)
                for j in range(EMB // L):
                    v = r[t, pl.ds(j * L, L)]
                    plsc.store_scatter(f, [ehi[j], elo[j], tv], v * SCALE)

        for b in range(nb):
            fire_gather(b, b)

        @pl.loop(0, n_s - nb, step=nb)
        def _(s0):
            for b in range(nb):
                drain_gather(s0 + b, b)
                transpose(b)
                fire_out(s0 + b, b)
                fire_gather(s0 + nb + b, b)
            for b in range(nb):
                drain_out(s0 + b, b)

        for b in range(nb):
            drain_gather(n_s - nb + b, b)
            transpose(b)
            fire_out(n_s - nb + b, b)
        for b in range(nb):
            drain_out(n_s - nb + b, b)

    return pl.kernel(
        body,
        out_type=jax.ShapeDtypeStruct((n_s, ET, NW, 8, CHUNK), jnp.float32),
        mesh=mesh,
        compiler_params=pltpu.CompilerParams(
            use_tc_tiling_on_sc=False, needs_layout_passes=False),
        scratch_types=(
            [pltpu.VMEM((CHUNK, n_s), jnp.int32),
             pltpu.VMEM((n_s, CHUNK), jnp.int32)]
            + [pltpu.VMEM((CHUNK, EMB), jnp.float32) for _ in range(nb)]
            + [pltpu.VMEM((ET, 8, CHUNK + 1), jnp.float32) for _ in range(nb)]
            + [pltpu.SemaphoreType.DMA for _ in range(2 * nb)]
        ),
    )


def kernel(tokens, table):
    bt, n_s = tokens.shape
    out5 = _make(n_s)(tokens.astype(jnp.int32), table)
    return out5.transpose(2, 4, 0, 1, 3).reshape(bt, n_s, EMB)
